# baseline (speedup 1.0000x reference)
import functools

import jax
import jax.numpy as jnp
from jax import lax
from jax.experimental import pallas as pl
from jax.experimental.pallas import tpu as pltpu
from jax.experimental.pallas import tpu_sc as plsc

N = 4096
C = 256
K = 16
L = 4

RB_KNN = 256
RB = 512


def _knn_body(pxc, pyc, pzc, pxr, pyr, pzr, out_ref):
    xc, yc, zc = pxc[...], pyc[...], pzc[...]
    xr, yr, zr = pxr[...], pyr[...], pzr[...]
    sq_c = xc * xc + yc * yc + zc * zc
    sq_r = xr * xr + yr * yr + zr * zr
    def _b(t):
        return t.astype(jnp.bfloat16).astype(jnp.float32)
    dot = _b(xc) * _b(xr) + _b(yc) * _b(yr) + _b(zc) * _b(zr)
    d = sq_c + sq_r - 2.0 * dot
    iota = lax.broadcasted_iota(jnp.int32, (RB_KNN, N), 1)
    cols = []
    for _ in range(K):
        m = jnp.min(d, axis=1, keepdims=True)
        am = jnp.min(jnp.where(d == m, iota, jnp.int32(N)), axis=1, keepdims=True)
        cols.append(am)
        d = jnp.where(iota == am, jnp.float32(jnp.inf), d)
    out_ref[...] = jnp.concatenate(cols, axis=1)


def _knn(p):
    px = p[:, 0:1]
    py = p[:, 1:2]
    pz = p[:, 2:3]
    grid = N // RB_KNN
    return pl.pallas_call(
        _knn_body,
        grid=(grid,),
        in_specs=[
            pl.BlockSpec((RB_KNN, 1), lambda i: (i, 0)),
            pl.BlockSpec((RB_KNN, 1), lambda i: (i, 0)),
            pl.BlockSpec((RB_KNN, 1), lambda i: (i, 0)),
            pl.BlockSpec((1, N), lambda i: (0, 0)),
            pl.BlockSpec((1, N), lambda i: (0, 0)),
            pl.BlockSpec((1, N), lambda i: (0, 0)),
        ],
        out_specs=pl.BlockSpec((RB_KNN, K), lambda i: (i, 0)),
        out_shape=jax.ShapeDtypeStruct((N, K), jnp.int32),
    )(px, py, pz, px.T, py.T, pz.T)


def _ln(x, g, b):
    m = jnp.mean(x, axis=-1, keepdims=True)
    v = jnp.mean((x - m) ** 2, axis=-1, keepdims=True)
    return (x - m) / jnp.sqrt(v + 1e-5) * g + b


def _qkv_body(x_ref, g_ref, b_ref, w_ref, q_ref, kv_ref):
    xn = _ln(x_ref[...], g_ref[...], b_ref[...])
    qkv = jnp.dot(xn, w_ref[...], preferred_element_type=jnp.float32)
    q_ref[...] = qkv[:, :C]
    kb = qkv[:, C:2 * C].astype(jnp.bfloat16).astype(jnp.float32)
    vb = qkv[:, 2 * C:].astype(jnp.bfloat16).astype(jnp.float32)
    kbits = lax.shift_right_logical(lax.bitcast_convert_type(kb, jnp.uint32), jnp.uint32(16))
    vbits = lax.bitcast_convert_type(vb, jnp.uint32) & jnp.uint32(0xFFFF0000)
    kv_ref[...] = lax.bitcast_convert_type(kbits | vbits, jnp.int32)


def _qkv(x, g, b, w):
    grid = N // RB
    outs = [jax.ShapeDtypeStruct((N, C), jnp.float32),
            jax.ShapeDtypeStruct((N, C), jnp.int32)]
    return pl.pallas_call(
        _qkv_body,
        grid=(grid,),
        in_specs=[
            pl.BlockSpec((RB, C), lambda i: (i, 0)),
            pl.BlockSpec((1, C), lambda i: (0, 0)),
            pl.BlockSpec((1, C), lambda i: (0, 0)),
            pl.BlockSpec((C, 3 * C), lambda i: (0, 0)),
        ],
        out_specs=[pl.BlockSpec((RB, C), lambda i: (i, 0)),
                   pl.BlockSpec((RB, C), lambda i: (i, 0))],
        out_shape=outs,
    )(x, g.reshape(1, C), b.reshape(1, C), w)


def _erf(x):
    a1, a2, a3 = 0.254829592, -0.284496736, 1.421413741
    a4, a5, p_ = -1.453152027, 1.061405429, 0.3275911
    s = jnp.sign(x)
    ax = jnp.abs(x)
    t = 1.0 / (1.0 + p_ * ax)
    y = 1.0 - (((((a5 * t + a4) * t) + a3) * t + a2) * t + a1) * t * jnp.exp(-ax * ax)
    return s * y


def _gelu(x):
    return 0.5 * x * (1.0 + _erf(x * 0.7071067811865476))


def _post_body(x_ref, a_ref, wp_ref, bp_ref, g2_ref, b2_ref,
               w1_ref, b1_ref, w2_ref, bb2_ref, o_ref):
    t = x_ref[...] + jnp.dot(a_ref[...], wp_ref[...],
                             preferred_element_type=jnp.float32) + bp_ref[...]
    xn2 = _ln(t, g2_ref[...], b2_ref[...])
    h = _gelu(jnp.dot(xn2, w1_ref[...], preferred_element_type=jnp.float32)
              + b1_ref[...])
    o_ref[...] = t + jnp.dot(h, w2_ref[...],
                             preferred_element_type=jnp.float32) + bb2_ref[...]


def _post(x, a, wp, bp, g2, b2, w1, b1, w2, bb2):
    grid = x.shape[0] // RB
    return pl.pallas_call(
        _post_body,
        grid=(grid,),
        in_specs=[
            pl.BlockSpec((RB, C), lambda i: (i, 0)),
            pl.BlockSpec((RB, C), lambda i: (i, 0)),
            pl.BlockSpec((C, C), lambda i: (0, 0)),
            pl.BlockSpec((1, C), lambda i: (0, 0)),
            pl.BlockSpec((1, C), lambda i: (0, 0)),
            pl.BlockSpec((1, C), lambda i: (0, 0)),
            pl.BlockSpec((C, 4 * C), lambda i: (0, 0)),
            pl.BlockSpec((1, 4 * C), lambda i: (0, 0)),
            pl.BlockSpec((4 * C, C), lambda i: (0, 0)),
            pl.BlockSpec((1, C), lambda i: (0, 0)),
        ],
        out_specs=pl.BlockSpec((RB, C), lambda i: (i, 0)),
        out_shape=jax.ShapeDtypeStruct((x.shape[0], C), jnp.float32),
    )(x, a, wp, bp.reshape(1, C), g2.reshape(1, C), b2.reshape(1, C),
      w1, b1.reshape(1, 4 * C), w2, bb2.reshape(1, C))


_SCALE = float(C) ** (-0.5)
_NW = 32
_PPW = N // _NW
_CH = 8
_NCHUNK = _PPW // _CH

RB_ATTN = 256


def _sc_gather_call(kv, idx3, npts):
    ppw = npts // _NW
    nchunk = ppw // _CH
    mesh = plsc.VectorSubcoreMesh(core_axis_name="c", subcore_axis_name="s")
    nc = 2

    @functools.partial(
        pl.kernel,
        out_type=jax.ShapeDtypeStruct((npts * K, C), jnp.int32),
        mesh=mesh,
        scratch_types=[
            pltpu.VMEM((nchunk, _CH * K), jnp.int32),
            pltpu.VMEM((_CH * K, C), jnp.int32),
            pltpu.VMEM((_CH * K, C), jnp.int32),
            pltpu.SemaphoreType.DMA,
            pltpu.SemaphoreType.DMA,
            pltpu.SemaphoreType.DMA,
            pltpu.SemaphoreType.DMA,
        ],
    )
    def body(kv_hbm, idx_hbm, gkv_hbm, idxs, buf0, buf1, gs0, gs1, ws0, ws1):
        wid = lax.axis_index("s") * nc + lax.axis_index("c")
        pltpu.sync_copy(idx_hbm.at[wid], idxs)
        bufs = (buf0, buf1)
        gsems = (gs0, gs1)
        wsems = (ws0, ws1)
        gcp = [None] * nchunk
        wcp = [None] * nchunk
        gcp[0] = pltpu.async_copy(kv_hbm.at[idxs.at[0]], bufs[0], gsems[0])
        for c in range(nchunk):
            b = c & 1
            if c + 1 < nchunk:
                nb = (c + 1) & 1
                if c - 1 >= 0:
                    wcp[c - 1].wait()
                gcp[c + 1] = pltpu.async_copy(
                    kv_hbm.at[idxs.at[c + 1]], bufs[nb], gsems[nb])
            gcp[c].wait()
            row0 = (wid * ppw + c * _CH) * K
            wcp[c] = pltpu.async_copy(
                bufs[b], gkv_hbm.at[pl.ds(row0, _CH * K)], wsems[b])
        wcp[nchunk - 1].wait()
        if nchunk >= 2:
            wcp[nchunk - 2].wait()

    return body(kv, idx3)


def _attn_body(q_ref, gkv_ref, o_ref):
    w = lax.bitcast_convert_type(gkv_ref[...], jnp.uint32)
    kg = lax.bitcast_convert_type(
        lax.shift_left(w, jnp.uint32(16)), jnp.float32).reshape(RB_ATTN, K, C)
    vg = lax.bitcast_convert_type(
        w & jnp.uint32(0xFFFF0000), jnp.float32).reshape(RB_ATTN, K, C)
    q3 = q_ref[...].reshape(RB_ATTN, 1, C)
    s = jnp.sum(kg * q3, axis=-1) * _SCALE
    s = s - jnp.max(s, axis=-1, keepdims=True)
    e = jnp.exp(s)
    attn = e / jnp.sum(e, axis=-1, keepdims=True)
    o_ref[...] = jnp.sum(attn.reshape(RB_ATTN, K, 1) * vg, axis=1)


def _attn_tc(q, gkv):
    grid = q.shape[0] // RB_ATTN
    return pl.pallas_call(
        _attn_body,
        grid=(grid,),
        in_specs=[
            pl.BlockSpec((RB_ATTN, C), lambda i: (i, 0)),
            pl.BlockSpec((RB_ATTN * K, C), lambda i: (i, 0)),
        ],
        out_specs=pl.BlockSpec((RB_ATTN, C), lambda i: (i, 0)),
        out_shape=jax.ShapeDtypeStruct((q.shape[0], C), jnp.float32),
    )(q, gkv)


def kernel(p, x, ln1_g, ln1_b, Wqkv, Wproj, bproj, ln2_g, ln2_b, W1, b1, W2, b2, o):
    idx = _knn(p)
    half = N // 2
    idxh = [idx[h * half:(h + 1) * half].reshape(_NW, half // (_NW * _CH), _CH * K)
            for h in range(2)]
    for i in range(L):
        xq, kv32 = _qkv(x, ln1_g[i], ln1_b[i], Wqkv[i])
        g0 = _sc_gather_call(kv32, idxh[0], half)
        g1 = _sc_gather_call(kv32, idxh[1], half)
        a0 = _attn_tc(xq[:half], g0)
        a1 = _attn_tc(xq[half:], g1)
        o0 = _post(x[:half], a0, Wproj[i], bproj[i], ln2_g[i], ln2_b[i],
                   W1[i], b1[i], W2[i], b2[i])
        o1 = _post(x[half:], a1, Wproj[i], bproj[i], ln2_g[i], ln2_b[i],
                   W1[i], b1[i], W2[i], b2[i])
        x = jnp.concatenate([o0, o1], axis=0)
    return x


# speedup vs baseline: 1.0599x; 1.0599x over previous
import functools

import jax
import jax.numpy as jnp
from jax import lax
from jax.experimental import pallas as pl
from jax.experimental.pallas import tpu as pltpu
from jax.experimental.pallas import tpu_sc as plsc

N = 4096
C = 256
K = 16
L = 4

RB_KNN = 256
RB = 512


def _knn_body(pxc, pyc, pzc, pxr, pyr, pzr, out_ref):
    xc, yc, zc = pxc[...], pyc[...], pzc[...]
    xr, yr, zr = pxr[...], pyr[...], pzr[...]
    sq_c = xc * xc + yc * yc + zc * zc
    sq_r = xr * xr + yr * yr + zr * zr
    def _b(t):
        return t.astype(jnp.bfloat16).astype(jnp.float32)
    dot = _b(xc) * _b(xr) + _b(yc) * _b(yr) + _b(zc) * _b(zr)
    d = sq_c + sq_r - 2.0 * dot
    iota = lax.broadcasted_iota(jnp.int32, (RB_KNN, N), 1)
    cols = []
    for _ in range(K):
        am = jnp.argmin(d, axis=1).astype(jnp.int32).reshape(RB_KNN, 1)
        cols.append(am)
        d = jnp.where(iota == am, jnp.float32(jnp.inf), d)
    out_ref[...] = jnp.concatenate(cols, axis=1)


def _knn(p):
    px = p[:, 0:1]
    py = p[:, 1:2]
    pz = p[:, 2:3]
    grid = N // RB_KNN
    return pl.pallas_call(
        _knn_body,
        grid=(grid,),
        in_specs=[
            pl.BlockSpec((RB_KNN, 1), lambda i: (i, 0)),
            pl.BlockSpec((RB_KNN, 1), lambda i: (i, 0)),
            pl.BlockSpec((RB_KNN, 1), lambda i: (i, 0)),
            pl.BlockSpec((1, N), lambda i: (0, 0)),
            pl.BlockSpec((1, N), lambda i: (0, 0)),
            pl.BlockSpec((1, N), lambda i: (0, 0)),
        ],
        out_specs=pl.BlockSpec((RB_KNN, K), lambda i: (i, 0)),
        out_shape=jax.ShapeDtypeStruct((N, K), jnp.int32),
    )(px, py, pz, px.T, py.T, pz.T)


def _ln(x, g, b):
    m = jnp.mean(x, axis=-1, keepdims=True)
    v = jnp.mean((x - m) ** 2, axis=-1, keepdims=True)
    return (x - m) / jnp.sqrt(v + 1e-5) * g + b


def _qkv_body(x_ref, g_ref, b_ref, w_ref, q_ref, kv_ref):
    xn = _ln(x_ref[...], g_ref[...], b_ref[...])
    qkv = jnp.dot(xn, w_ref[...], preferred_element_type=jnp.float32)
    q_ref[...] = qkv[:, :C]
    kb = qkv[:, C:2 * C].astype(jnp.bfloat16).astype(jnp.float32)
    vb = qkv[:, 2 * C:].astype(jnp.bfloat16).astype(jnp.float32)
    kbits = lax.shift_right_logical(lax.bitcast_convert_type(kb, jnp.uint32), jnp.uint32(16))
    vbits = lax.bitcast_convert_type(vb, jnp.uint32) & jnp.uint32(0xFFFF0000)
    kv_ref[...] = lax.bitcast_convert_type(kbits | vbits, jnp.int32)


def _qkv(x, g, b, w):
    grid = N // RB
    outs = [jax.ShapeDtypeStruct((N, C), jnp.float32),
            jax.ShapeDtypeStruct((N, C), jnp.int32)]
    return pl.pallas_call(
        _qkv_body,
        grid=(grid,),
        in_specs=[
            pl.BlockSpec((RB, C), lambda i: (i, 0)),
            pl.BlockSpec((1, C), lambda i: (0, 0)),
            pl.BlockSpec((1, C), lambda i: (0, 0)),
            pl.BlockSpec((C, 3 * C), lambda i: (0, 0)),
        ],
        out_specs=[pl.BlockSpec((RB, C), lambda i: (i, 0)),
                   pl.BlockSpec((RB, C), lambda i: (i, 0))],
        out_shape=outs,
    )(x, g.reshape(1, C), b.reshape(1, C), w)


def _erf(x):
    a1, a2, a3 = 0.254829592, -0.284496736, 1.421413741
    a4, a5, p_ = -1.453152027, 1.061405429, 0.3275911
    s = jnp.sign(x)
    ax = jnp.abs(x)
    t = 1.0 / (1.0 + p_ * ax)
    y = 1.0 - (((((a5 * t + a4) * t) + a3) * t + a2) * t + a1) * t * jnp.exp(-ax * ax)
    return s * y


def _gelu(x):
    return 0.5 * x * (1.0 + _erf(x * 0.7071067811865476))


def _post_body(x_ref, a_ref, wp_ref, bp_ref, g2_ref, b2_ref,
               w1_ref, b1_ref, w2_ref, bb2_ref, o_ref):
    t = x_ref[...] + jnp.dot(a_ref[...], wp_ref[...],
                             preferred_element_type=jnp.float32) + bp_ref[...]
    xn2 = _ln(t, g2_ref[...], b2_ref[...])
    h = _gelu(jnp.dot(xn2, w1_ref[...], preferred_element_type=jnp.float32)
              + b1_ref[...])
    o_ref[...] = t + jnp.dot(h, w2_ref[...],
                             preferred_element_type=jnp.float32) + bb2_ref[...]


def _post(x, a, wp, bp, g2, b2, w1, b1, w2, bb2):
    grid = x.shape[0] // RB
    return pl.pallas_call(
        _post_body,
        grid=(grid,),
        in_specs=[
            pl.BlockSpec((RB, C), lambda i: (i, 0)),
            pl.BlockSpec((RB, C), lambda i: (i, 0)),
            pl.BlockSpec((C, C), lambda i: (0, 0)),
            pl.BlockSpec((1, C), lambda i: (0, 0)),
            pl.BlockSpec((1, C), lambda i: (0, 0)),
            pl.BlockSpec((1, C), lambda i: (0, 0)),
            pl.BlockSpec((C, 4 * C), lambda i: (0, 0)),
            pl.BlockSpec((1, 4 * C), lambda i: (0, 0)),
            pl.BlockSpec((4 * C, C), lambda i: (0, 0)),
            pl.BlockSpec((1, C), lambda i: (0, 0)),
        ],
        out_specs=pl.BlockSpec((RB, C), lambda i: (i, 0)),
        out_shape=jax.ShapeDtypeStruct((x.shape[0], C), jnp.float32),
    )(x, a, wp, bp.reshape(1, C), g2.reshape(1, C), b2.reshape(1, C),
      w1, b1.reshape(1, 4 * C), w2, bb2.reshape(1, C))


_SCALE = float(C) ** (-0.5)
_NW = 32
_PPW = N // _NW
_CH = 8
_NCHUNK = _PPW // _CH

RB_ATTN = 256


def _sc_gather_call(kv, idx3, npts):
    ppw = npts // _NW
    nchunk = ppw // _CH
    mesh = plsc.VectorSubcoreMesh(core_axis_name="c", subcore_axis_name="s")
    nc = 2

    @functools.partial(
        pl.kernel,
        out_type=jax.ShapeDtypeStruct((npts * K, C), jnp.int32),
        mesh=mesh,
        scratch_types=[
            pltpu.VMEM((nchunk, _CH * K), jnp.int32),
            pltpu.VMEM((_CH * K, C), jnp.int32),
            pltpu.VMEM((_CH * K, C), jnp.int32),
            pltpu.SemaphoreType.DMA,
            pltpu.SemaphoreType.DMA,
            pltpu.SemaphoreType.DMA,
            pltpu.SemaphoreType.DMA,
        ],
    )
    def body(kv_hbm, idx_hbm, gkv_hbm, idxs, buf0, buf1, gs0, gs1, ws0, ws1):
        wid = lax.axis_index("s") * nc + lax.axis_index("c")
        pltpu.sync_copy(idx_hbm.at[wid], idxs)
        bufs = (buf0, buf1)
        gsems = (gs0, gs1)
        wsems = (ws0, ws1)
        gcp = [None] * nchunk
        wcp = [None] * nchunk
        gcp[0] = pltpu.async_copy(kv_hbm.at[idxs.at[0]], bufs[0], gsems[0])
        for c in range(nchunk):
            b = c & 1
            if c + 1 < nchunk:
                nb = (c + 1) & 1
                if c - 1 >= 0:
                    wcp[c - 1].wait()
                gcp[c + 1] = pltpu.async_copy(
                    kv_hbm.at[idxs.at[c + 1]], bufs[nb], gsems[nb])
            gcp[c].wait()
            row0 = (wid * ppw + c * _CH) * K
            wcp[c] = pltpu.async_copy(
                bufs[b], gkv_hbm.at[pl.ds(row0, _CH * K)], wsems[b])
        wcp[nchunk - 1].wait()
        if nchunk >= 2:
            wcp[nchunk - 2].wait()

    return body(kv, idx3)


def _attn_body(q_ref, gkv_ref, o_ref):
    w = lax.bitcast_convert_type(gkv_ref[...], jnp.uint32)
    kg = lax.bitcast_convert_type(
        lax.shift_left(w, jnp.uint32(16)), jnp.float32).reshape(RB_ATTN, K, C)
    vg = lax.bitcast_convert_type(
        w & jnp.uint32(0xFFFF0000), jnp.float32).reshape(RB_ATTN, K, C)
    q3 = q_ref[...].reshape(RB_ATTN, 1, C)
    s = jnp.sum(kg * q3, axis=-1) * _SCALE
    s = s - jnp.max(s, axis=-1, keepdims=True)
    e = jnp.exp(s)
    attn = e / jnp.sum(e, axis=-1, keepdims=True)
    o_ref[...] = jnp.sum(attn.reshape(RB_ATTN, K, 1) * vg, axis=1)


def _attn_tc(q, gkv):
    grid = q.shape[0] // RB_ATTN
    return pl.pallas_call(
        _attn_body,
        grid=(grid,),
        in_specs=[
            pl.BlockSpec((RB_ATTN, C), lambda i: (i, 0)),
            pl.BlockSpec((RB_ATTN * K, C), lambda i: (i, 0)),
        ],
        out_specs=pl.BlockSpec((RB_ATTN, C), lambda i: (i, 0)),
        out_shape=jax.ShapeDtypeStruct((q.shape[0], C), jnp.float32),
    )(q, gkv)


def kernel(p, x, ln1_g, ln1_b, Wqkv, Wproj, bproj, ln2_g, ln2_b, W1, b1, W2, b2, o):
    idx = _knn(p)
    idx3 = idx.reshape(_NW, N // (_NW * _CH), _CH * K)
    for i in range(L):
        xq, kv32 = _qkv(x, ln1_g[i], ln1_b[i], Wqkv[i])
        gkv32 = _sc_gather_call(kv32, idx3, N)
        a = _attn_tc(xq, gkv32)
        x = _post(x, a, Wproj[i], bproj[i], ln2_g[i], ln2_b[i],
                  W1[i], b1[i], W2[i], b2[i])
    return x
